# baseline (speedup 1.0000x reference)
import functools

import jax
import jax.numpy as jnp
from jax import lax
from jax.experimental import pallas as pl
from jax.experimental.pallas import tpu as pltpu
from jax.experimental.pallas import tpu_sc as plsc

_NC = 2
_NS = 16
_NW = _NC * _NS

_N_NODES = 10000
_N_ROWS = 320000
_D = 128
_B_PER_W = _N_ROWS // _NW
_CHUNK = 104
_NBUF = 3
_N_CHUNKS = _B_PER_W // _CHUNK
_TAIL = _B_PER_W - _N_CHUNKS * _CHUNK
_ROWS_PER_TILE = 624
_STAGE_TAIL = _N_NODES - _ROWS_PER_TILE * _NS


def _gather_kernel(x_hbm, idx_hbm, out_hbm, x_sh, idx_v, rows_v, sems,
                   stg_sem, idx_sem, tail_sem):
    sid = lax.axis_index("s")
    wid = sid * _NC + lax.axis_index("c")
    base = wid * _B_PER_W

    stg = pltpu.async_copy(
        x_hbm.at[pl.ds(sid * _ROWS_PER_TILE, _ROWS_PER_TILE)],
        x_sh.at[pl.ds(sid * _ROWS_PER_TILE, _ROWS_PER_TILE)], stg_sem)
    idx_cp = pltpu.async_copy(idx_hbm.at[pl.ds(base, _B_PER_W)], idx_v,
                              idx_sem)

    @pl.when(sid == 0)
    def _():
        pltpu.async_copy(x_hbm.at[pl.ds(_ROWS_PER_TILE * _NS, _STAGE_TAIL)],
                         x_sh.at[pl.ds(_ROWS_PER_TILE * _NS, _STAGE_TAIL)],
                         tail_sem)

    def _start(g, buf, nrows=_CHUNK, src=None):
        pltpu.async_copy(
            (x_sh if src is None else src).at[idx_v.at[pl.ds(g * _CHUNK,
                                                             nrows)]],
            rows_v.at[buf].at[pl.ds(0, nrows)],
            sems.at[buf],
        )

    def _finish(g, buf, nrows=_CHUNK, src=None):
        pltpu.make_async_copy(
            (x_sh if src is None else src).at[idx_v.at[pl.ds(g * _CHUNK,
                                                             nrows)]],
            rows_v.at[buf].at[pl.ds(0, nrows)],
            sems.at[buf],
        ).wait()
        pltpu.sync_copy(rows_v.at[buf].at[pl.ds(0, nrows)],
                        out_hbm.at[pl.ds(base + g * _CHUNK, nrows)])

    idx_cp.wait()
    for b in range(_NBUF):
        _start(b, b, src=x_hbm)

    stg.wait()

    @pl.when(sid == 0)
    def _():
        pltpu.make_async_copy(
            x_hbm.at[pl.ds(_ROWS_PER_TILE * _NS, _STAGE_TAIL)],
            x_sh.at[pl.ds(_ROWS_PER_TILE * _NS, _STAGE_TAIL)],
            tail_sem).wait()
    plsc.subcore_barrier()

    for b in range(_NBUF):
        _finish(b, b, src=x_hbm)
        _start(b + _NBUF, b)

    def body(i, _):
        g = _NBUF + i * _NBUF
        for b in range(_NBUF):
            _finish(g + b, b)
            _start(g + b + _NBUF, b)
        return _

    lax.fori_loop(0, (_N_CHUNKS - 2 * _NBUF) // _NBUF, body, None)
    _finish(_N_CHUNKS - _NBUF, (_N_CHUNKS - _NBUF) % _NBUF)
    _start(_N_CHUNKS, _N_CHUNKS % _NBUF, _TAIL)
    for g in range(_N_CHUNKS - _NBUF + 1, _N_CHUNKS):
        _finish(g, g % _NBUF)
    _finish(_N_CHUNKS, _N_CHUNKS % _NBUF, _TAIL)


@jax.jit
def _gather(x, index):
    run = pl.kernel(
        _gather_kernel,
        out_type=jax.ShapeDtypeStruct((_N_ROWS, _D), jnp.float32),
        mesh=plsc.VectorSubcoreMesh(core_axis_name="c", subcore_axis_name="s",
                                    num_cores=_NC, num_subcores=_NS),
        scratch_types=[
            pltpu.VMEM_SHARED((_N_NODES, _D), jnp.float32),
            pltpu.VMEM((_B_PER_W,), jnp.int32),
            pltpu.VMEM((_NBUF, _CHUNK, _D), jnp.float32),
            pltpu.SemaphoreType.DMA((_NBUF,)),
            pltpu.SemaphoreType.DMA,
            pltpu.SemaphoreType.DMA,
            pltpu.SemaphoreType.DMA,
        ],
    )
    return run(x, index)


def kernel(x, index):
    return _gather(x, jnp.reshape(index, (-1,)).astype(jnp.int32))
